# baseline (speedup 1.0000x reference)
import functools

import jax
import jax.numpy as jnp
import numpy as np
from jax.experimental import pallas as pl
from jax.experimental.pallas import tpu as pltpu

_FACTOR = 5


_U32 = np.uint32


def _threefry2x32_np(k1, k2, x0, x1):
    def rotl(x, d):
        return ((x << _U32(d)) | (x >> _U32(32 - d))).astype(np.uint32)

    ks0, ks1 = _U32(k1), _U32(k2)
    ks2 = _U32(ks0 ^ ks1 ^ _U32(0x1BD11BDA))
    x = [(x0 + ks0).astype(np.uint32), (x1 + ks1).astype(np.uint32)]
    r0 = (13, 15, 26, 6)
    r1 = (17, 29, 16, 24)

    def rounds(x, rs):
        for r in rs:
            x[0] = (x[0] + x[1]).astype(np.uint32)
            x[1] = x[0] ^ rotl(x[1], r)
        return x

    for i, (rs, ka, kb) in enumerate(
        [(r0, ks1, ks2), (r1, ks2, ks0), (r0, ks0, ks1),
         (r1, ks1, ks2), (r0, ks2, ks0)]):
        x = rounds(x, rs)
        x[0] = (x[0] + ka).astype(np.uint32)
        x[1] = (x[1] + kb + _U32(i + 1)).astype(np.uint32)
    return x


def _iota_2x32(shape):
    n = int(np.prod(shape))
    counts = np.arange(n, dtype=np.uint64)
    hi = (counts >> np.uint64(32)).astype(np.uint32).reshape(shape)
    lo = (counts & np.uint64(0xFFFFFFFF)).astype(np.uint32).reshape(shape)
    return hi, lo


def _randint_np(seed, shape, minval, maxval):
    key = (_U32(np.uint64(seed) >> np.uint64(32)),
           _U32(np.uint64(seed) & np.uint64(0xFFFFFFFF)))
    hi, lo = _iota_2x32((2,))
    b1, b2 = _threefry2x32_np(key[0], key[1], hi, lo)
    k1 = (b1[0], b2[0])
    k2 = (b1[1], b2[1])
    hi, lo = _iota_2x32(shape)
    hb1, hb2 = _threefry2x32_np(k1[0], k1[1], hi, lo)
    higher_bits = hb1 ^ hb2
    lb1, lb2 = _threefry2x32_np(k2[0], k2[1], hi, lo)
    lower_bits = lb1 ^ lb2
    span = _U32(maxval - minval)
    multiplier = _U32(pow(2, 16, int(span)))
    multiplier = _U32((int(multiplier) * int(multiplier)) % int(span))
    offset = ((higher_bits % span) * multiplier + lower_bits % span) % span
    return (np.int64(minval) + offset.astype(np.int64)).astype(np.int32)


@functools.lru_cache(maxsize=4)
def _sample_masks(l_q: int, l_k: int):
    u_part = min(int(_FACTOR * np.ceil(np.log(l_k))), l_k)
    idx = _randint_np(123, (l_q, u_part), 0, l_k)
    cnt_t = np.zeros((l_k, l_q), dtype=np.float32)
    np.add.at(cnt_t, (idx, np.arange(l_q)[:, None]), 1.0)
    madd_t = np.where(cnt_t > 0.0, np.float32(0.0),
                      np.float32(-np.inf)).astype(np.float32)
    return cnt_t, madd_t


def _head_kernel(q_ref, qd_ref, k_ref, v_ref, cnt_ref, madd_ref, out_ref,
                 qsel_ref, upd_ref, idx_ref, *, u: int):
    l_q, d = q_ref.shape[1], q_ref.shape[2]
    l_k = k_ref.shape[1]
    q = q_ref[0]
    k = k_ref[0]
    v = v_ref[0]
    qb = q.astype(jnp.bfloat16)
    kb = k.astype(jnp.bfloat16)
    st = jax.lax.dot_general(
        kb, qb, (((1,), (1,)), ((), ())),
        preferred_element_type=jnp.float32)
    kc = jax.lax.dot_general(
        kb, cnt_ref[...], (((0,), (0,)), ((), ())),
        preferred_element_type=jnp.float32)
    ssum = jnp.sum(qd_ref[0].astype(jnp.float32) * kc, axis=0, keepdims=True)
    smax = jnp.max(st + madd_ref[...], axis=0, keepdims=True)
    m = smax - ssum / np.float32(l_k)
    laneid = jax.lax.broadcasted_iota(jnp.int32, (1, l_q), 1)

    def select_body(i, m_cur):
        val = jnp.max(m_cur)
        idx = jnp.min(jnp.where(m_cur == val, laneid, l_q))
        idx_ref[i] = idx
        qsel_ref[pl.ds(i, 1), :] = q_ref[0, pl.ds(idx, 1), :]
        return jnp.where(laneid == idx, -jnp.inf, m_cur)

    jax.lax.fori_loop(0, u, select_body, m)

    scale = np.float32(1.0 / np.sqrt(d))
    rows = jax.lax.dot_general(
        qsel_ref[...].astype(jnp.bfloat16), kb,
        (((1,), (1,)), ((), ())),
        preferred_element_type=jnp.float32) * scale
    rows = rows - jnp.max(rows, axis=1, keepdims=True)
    e = jnp.exp(rows)
    attn = e / jnp.sum(e, axis=1, keepdims=True)
    upd_ref[...] = jax.lax.dot_general(
        attn.astype(jnp.bfloat16), v.astype(jnp.bfloat16),
        (((1,), (0,)), ((), ())),
        preferred_element_type=jnp.float32)

    vmean = jnp.mean(v, axis=0, keepdims=True)
    out_ref[0] = jnp.broadcast_to(vmean, (l_q, d))

    def scatter_body(i, carry):
        out_ref[0, pl.ds(idx_ref[i], 1), :] = upd_ref[pl.ds(i, 1), :]
        return carry

    jax.lax.fori_loop(0, u, scatter_body, 0)


def kernel(queries, keys, values, attn_mask):
    b, l_q, h, d = queries.shape
    l_k = keys.shape[1]
    u = min(int(_FACTOR * np.ceil(np.log(l_q))), l_q)
    cnt_t, madd_t = _sample_masks(l_q, l_k)
    cnt_t = jnp.asarray(cnt_t).astype(jnp.bfloat16)
    madd_t = jnp.asarray(madd_t)

    qt = jnp.transpose(queries, (0, 2, 1, 3)).reshape(b * h, l_q, d)
    kt = jnp.transpose(keys, (0, 2, 1, 3)).reshape(b * h, l_k, d)
    vt = jnp.transpose(values, (0, 2, 1, 3)).reshape(b * h, l_k, d)
    qdt = jnp.transpose(queries.astype(jnp.bfloat16),
                        (0, 2, 3, 1)).reshape(b * h, d, l_q)

    out = pl.pallas_call(
        functools.partial(_head_kernel, u=u),
        grid=(b * h,),
        in_specs=[
            pl.BlockSpec((1, l_q, d), lambda i: (i, 0, 0)),
            pl.BlockSpec((1, d, l_q), lambda i: (i, 0, 0)),
            pl.BlockSpec((1, l_k, d), lambda i: (i, 0, 0)),
            pl.BlockSpec((1, l_k, d), lambda i: (i, 0, 0)),
            pl.BlockSpec((l_k, l_q), lambda i: (0, 0)),
            pl.BlockSpec((l_k, l_q), lambda i: (0, 0)),
        ],
        out_specs=pl.BlockSpec((1, l_q, d), lambda i: (i, 0, 0)),
        out_shape=jax.ShapeDtypeStruct((b * h, l_q, d), jnp.float32),
        scratch_shapes=[
            pltpu.VMEM((u, d), jnp.float32),
            pltpu.VMEM((u, d), jnp.float32),
            pltpu.SMEM((u,), jnp.int32),
        ],
        compiler_params=pltpu.CompilerParams(
            dimension_semantics=("arbitrary",),
        ),
    )(qt, qdt, kt, vt, cnt_t, madd_t)

    out = out.reshape(b, h, l_q, d)
    return jnp.transpose(out, (0, 2, 1, 3))


# speedup vs baseline: 2.3301x; 2.3301x over previous
import functools

import jax
import jax.numpy as jnp
import numpy as np
from jax.experimental import pallas as pl
from jax.experimental.pallas import tpu as pltpu

_FACTOR = 5

_U32 = np.uint32


def _threefry2x32_np(k1, k2, x0, x1):
    def rotl(x, d):
        return ((x << _U32(d)) | (x >> _U32(32 - d))).astype(np.uint32)

    ks0, ks1 = _U32(k1), _U32(k2)
    ks2 = _U32(ks0 ^ ks1 ^ _U32(0x1BD11BDA))
    x = [(x0 + ks0).astype(np.uint32), (x1 + ks1).astype(np.uint32)]
    r0 = (13, 15, 26, 6)
    r1 = (17, 29, 16, 24)

    def rounds(x, rs):
        for r in rs:
            x[0] = (x[0] + x[1]).astype(np.uint32)
            x[1] = x[0] ^ rotl(x[1], r)
        return x

    for i, (rs, ka, kb) in enumerate(
        [(r0, ks1, ks2), (r1, ks2, ks0), (r0, ks0, ks1),
         (r1, ks1, ks2), (r0, ks2, ks0)]):
        x = rounds(x, rs)
        x[0] = (x[0] + ka).astype(np.uint32)
        x[1] = (x[1] + kb + _U32(i + 1)).astype(np.uint32)
    return x


def _iota_2x32(shape):
    n = int(np.prod(shape))
    counts = np.arange(n, dtype=np.uint64)
    hi = (counts >> np.uint64(32)).astype(np.uint32).reshape(shape)
    lo = (counts & np.uint64(0xFFFFFFFF)).astype(np.uint32).reshape(shape)
    return hi, lo


def _randint_np(seed, shape, minval, maxval):
    key = (_U32(np.uint64(seed) >> np.uint64(32)),
           _U32(np.uint64(seed) & np.uint64(0xFFFFFFFF)))
    hi, lo = _iota_2x32((2,))
    b1, b2 = _threefry2x32_np(key[0], key[1], hi, lo)
    k1 = (b1[0], b2[0])
    k2 = (b1[1], b2[1])
    hi, lo = _iota_2x32(shape)
    hb1, hb2 = _threefry2x32_np(k1[0], k1[1], hi, lo)
    higher_bits = hb1 ^ hb2
    lb1, lb2 = _threefry2x32_np(k2[0], k2[1], hi, lo)
    lower_bits = lb1 ^ lb2
    span = _U32(maxval - minval)
    multiplier = _U32(pow(2, 16, int(span)))
    multiplier = _U32((int(multiplier) * int(multiplier)) % int(span))
    offset = ((higher_bits % span) * multiplier + lower_bits % span) % span
    return (np.int64(minval) + offset.astype(np.int64)).astype(np.int32)


@functools.lru_cache(maxsize=4)
def _sample_masks(l_q: int, l_k: int):
    u_part = min(int(_FACTOR * np.ceil(np.log(l_k))), l_k)
    idx = _randint_np(123, (l_q, u_part), 0, l_k)
    cnt_t = np.zeros((l_k, l_q), dtype=np.float32)
    np.add.at(cnt_t, (idx, np.arange(l_q)[:, None]), 1.0)
    madd_t = np.where(cnt_t > 0.0, np.float32(0.0),
                      np.float32(-np.inf)).astype(np.float32)
    return cnt_t, madd_t


def _measure_kernel(q_ref, k_ref, cnt_ref, madd_ref, m_ref):
    l_k = k_ref.shape[1]
    qb = q_ref[0].astype(jnp.bfloat16)
    kb = k_ref[0].astype(jnp.bfloat16)
    st = jax.lax.dot_general(
        kb, qb, (((1,), (1,)), ((), ())),
        preferred_element_type=jnp.float32)
    ssum = jnp.sum(st * cnt_ref[...], axis=0, keepdims=True)
    smax = jnp.max(st + madd_ref[...], axis=0, keepdims=True)
    m_ref[0] = smax - ssum / np.float32(l_k)


def _select_kernel(m_ref, idx_ref, *, u: int):
    h, l_q = m_ref.shape[0], m_ref.shape[2]
    m = m_ref[:, 0, :]
    laneid = jax.lax.broadcasted_iota(jnp.int32, (h, l_q), 1)
    colid = jax.lax.broadcasted_iota(jnp.int32, (h, u), 1)
    idx0 = jnp.zeros((h, u), jnp.int32)

    def body(i, carry):
        m_cur, idxmat = carry
        val = jnp.max(m_cur, axis=1, keepdims=True)
        idx = jnp.min(jnp.where(m_cur == val, laneid, l_q),
                      axis=1, keepdims=True)
        idxmat = jnp.where(colid == i, idx, idxmat)
        m_cur = jnp.where(laneid == idx, -jnp.inf, m_cur)
        return m_cur, idxmat

    _, idxmat = jax.lax.fori_loop(0, u, body, (m, idx0))
    idx_ref[...] = idxmat


def _attend_kernel(idx_sref, q_ref, k_ref, v_ref, out_ref,
                   qsel_ref, upd_ref, *, u: int):
    l_q, d = q_ref.shape[1], q_ref.shape[2]
    h = pl.program_id(0)
    kb = k_ref[0].astype(jnp.bfloat16)
    v = v_ref[0]

    def gather_body(i, carry):
        qsel_ref[pl.ds(i, 1), :] = q_ref[0, pl.ds(idx_sref[h, i], 1), :]
        return carry

    jax.lax.fori_loop(0, u, gather_body, 0)

    scale = np.float32(1.0 / np.sqrt(d))
    rows = jax.lax.dot_general(
        qsel_ref[...].astype(jnp.bfloat16), kb,
        (((1,), (1,)), ((), ())),
        preferred_element_type=jnp.float32) * scale
    rows = rows - jnp.max(rows, axis=1, keepdims=True)
    e = jnp.exp(rows)
    attn = e / jnp.sum(e, axis=1, keepdims=True)
    upd_ref[...] = jax.lax.dot_general(
        attn.astype(jnp.bfloat16), v.astype(jnp.bfloat16),
        (((1,), (0,)), ((), ())),
        preferred_element_type=jnp.float32)

    vmean = jnp.mean(v, axis=0, keepdims=True)
    out_ref[0] = jnp.broadcast_to(vmean, (l_q, d))

    def scatter_body(i, carry):
        out_ref[0, pl.ds(idx_sref[h, i], 1), :] = upd_ref[pl.ds(i, 1), :]
        return carry

    jax.lax.fori_loop(0, u, scatter_body, 0)


def kernel(queries, keys, values, attn_mask):
    b, l_q, h, d = queries.shape
    l_k = keys.shape[1]
    u = min(int(_FACTOR * np.ceil(np.log(l_q))), l_q)
    cnt_t, madd_t = _sample_masks(l_q, l_k)
    cnt_t = jnp.asarray(cnt_t)
    madd_t = jnp.asarray(madd_t)

    qt = jnp.transpose(queries, (0, 2, 1, 3)).reshape(b * h, l_q, d)
    kt = jnp.transpose(keys, (0, 2, 1, 3)).reshape(b * h, l_k, d)
    vt = jnp.transpose(values, (0, 2, 1, 3)).reshape(b * h, l_k, d)
    bh = b * h

    m_all = pl.pallas_call(
        _measure_kernel,
        grid=(bh,),
        in_specs=[
            pl.BlockSpec((1, l_q, d), lambda i: (i, 0, 0)),
            pl.BlockSpec((1, l_k, d), lambda i: (i, 0, 0)),
            pl.BlockSpec((l_k, l_q), lambda i: (0, 0)),
            pl.BlockSpec((l_k, l_q), lambda i: (0, 0)),
        ],
        out_specs=pl.BlockSpec((1, 1, l_q), lambda i: (i, 0, 0)),
        out_shape=jax.ShapeDtypeStruct((bh, 1, l_q), jnp.float32),
        compiler_params=pltpu.CompilerParams(
            dimension_semantics=("arbitrary",),
        ),
    )(qt, kt, cnt_t, madd_t)

    idx_all = pl.pallas_call(
        functools.partial(_select_kernel, u=u),
        in_specs=[pl.BlockSpec((bh, 1, l_q), lambda: (0, 0, 0))],
        out_specs=pl.BlockSpec((bh, u), lambda: (0, 0)),
        out_shape=jax.ShapeDtypeStruct((bh, u), jnp.int32),
    )(m_all)

    out = pl.pallas_call(
        functools.partial(_attend_kernel, u=u),
        grid_spec=pltpu.PrefetchScalarGridSpec(
            num_scalar_prefetch=1,
            grid=(bh,),
            in_specs=[
                pl.BlockSpec((1, l_q, d), lambda i, idx: (i, 0, 0)),
                pl.BlockSpec((1, l_k, d), lambda i, idx: (i, 0, 0)),
                pl.BlockSpec((1, l_k, d), lambda i, idx: (i, 0, 0)),
            ],
            out_specs=pl.BlockSpec((1, l_q, d), lambda i, idx: (i, 0, 0)),
            scratch_shapes=[
                pltpu.VMEM((u, d), jnp.float32),
                pltpu.VMEM((u, d), jnp.float32),
            ],
        ),
        out_shape=jax.ShapeDtypeStruct((bh, l_q, d), jnp.float32),
        compiler_params=pltpu.CompilerParams(
            dimension_semantics=("arbitrary",),
        ),
    )(idx_all, qt, kt, vt)

    out = out.reshape(b, h, l_q, d)
    return jnp.transpose(out, (0, 2, 1, 3))
